# baseline (speedup 1.0000x reference)
import jax
import jax.numpy as jnp
from jax.experimental import pallas as pl

_MEDNUM = 2048
_FDIM = 64
_BR = 256


def _ddi_gcn_kernel(adj_top_ref, adj_bot_ref, w_ref, inter_ref, out_ref):
    w = w_ref[:]
    a1 = adj_top_ref[:, :_MEDNUM] + adj_top_ref[:, _MEDNUM:]
    a2 = adj_bot_ref[:, :_MEDNUM] + adj_bot_ref[:, _MEDNUM:]
    y1 = jnp.maximum(jnp.dot(a1, w, preferred_element_type=jnp.float32), 0.0)
    y2 = jnp.maximum(jnp.dot(a2, w, preferred_element_type=jnp.float32), 0.0)
    t = inter_ref[0, 0]
    out_ref[:] = (2.0 * t) * y1 + (2.0 - 2.0 * t) * y2


@jax.jit
def kernel(adj1, mEmbed, inter):
    n_tiles = _MEDNUM // _BR
    return pl.pallas_call(
        _ddi_gcn_kernel,
        grid=(n_tiles,),
        in_specs=[
            pl.BlockSpec((_BR, 2 * _MEDNUM), lambda j: (j, 0)),
            pl.BlockSpec((_BR, 2 * _MEDNUM), lambda j: (j + _MEDNUM // _BR, 0)),
            pl.BlockSpec((_MEDNUM, _FDIM), lambda j: (0, 0)),
            pl.BlockSpec((1, 1), lambda j: (0, 0)),
        ],
        out_specs=pl.BlockSpec((_BR, _FDIM), lambda j: (j, 0)),
        out_shape=jax.ShapeDtypeStruct((_MEDNUM, _FDIM), jnp.float32),
    )(adj1, adj1, mEmbed, inter.reshape(1, 1))
